# Initial kernel scaffold; baseline (speedup 1.0000x reference)
import jax
import jax.numpy as jnp
from jax.experimental import pallas as pl


def kernel(X, vertex, edges, alpha, beta, X0, degE, degV, W):
    raise NotImplementedError("write your pallas kernel here")


# speedup vs baseline: 7.8248x; 7.8248x over previous
import functools

import jax
import jax.numpy as jnp
from jax import lax
from jax.experimental import pallas as pl
from jax.experimental.pallas import tpu as pltpu
from jax.experimental.pallas import tpu_sc as plsc

N = 10000
E = 10000
NNZ = 320000
D = 128

NC = 2
NS = 16
NW = NC * NS
L = 16
CH = 128
NCHUNKS = 2560
NCH_W = NCHUNKS // NW
PAD_NNZ = NCHUNKS * CH
EP = 10240
RPS = EP // NS
IB = 8

_f32 = jnp.float32


def _make_hop_body(do_count):
    def _hop_body(src, gidx, sidx, zrow, zhist, out_sum, out_cnt,
                  gi_v, si_v, buf, hist, acc, sem):
        c = lax.axis_index("c")
        s = lax.axis_index("s")
        w = c * NS + s
        pltpu.sync_copy(zrow, buf)
        for t in range(RPS // CH):
            pltpu.sync_copy(buf, acc.at[pl.ds(s * RPS + t * CH, CH)])
        if do_count:
            pltpu.sync_copy(zhist, hist)
        plsc.subcore_barrier()

        def outer(b, carry):
            base = w * NCH_W + b * IB
            pltpu.sync_copy(gidx.at[pl.ds(base, IB)], gi_v)
            pltpu.sync_copy(sidx.at[pl.ds(base, IB)], si_v)

            def body(j, carry2):
                pltpu.async_copy(src.at[gi_v.at[j]], buf, sem).wait()
                pltpu.sync_copy(buf, acc.at[si_v.at[j]], add=True)
                if do_count:
                    for k in range(CH // L):
                        idx = si_v[j, pl.ds(k * L, L)]
                        cnts, last = plsc.scan_count(idx)
                        plsc.addupdate_scatter(hist, [idx], cnts, mask=last)
                return carry2

            lax.fori_loop(0, IB, body, 0)
            return carry

        lax.fori_loop(0, NCH_W // IB, outer, 0)
        plsc.subcore_barrier()
        for t in range(RPS // CH):
            r0 = s * RPS + t * CH
            pltpu.sync_copy(acc.at[pl.ds(r0, CH)], buf)
            pltpu.sync_copy(buf, out_sum.at[c, pl.ds(r0, CH)])
        if do_count:
            pltpu.sync_copy(hist, out_cnt.at[c, s])

    return _hop_body


@functools.cache
def _build_hop(do_count):
    cnt_rows = EP if do_count else 8
    return pl.kernel(
        _make_hop_body(do_count),
        out_type=(jax.ShapeDtypeStruct((NC, EP, D), _f32),
                  jax.ShapeDtypeStruct((NC, NS, cnt_rows), jnp.int32)),
        mesh=plsc.VectorSubcoreMesh(core_axis_name="c", subcore_axis_name="s",
                                    num_cores=NC, num_subcores=NS),
        scratch_types=[
            pltpu.VMEM((IB, CH), jnp.int32),
            pltpu.VMEM((IB, CH), jnp.int32),
            pltpu.VMEM((CH, D), _f32),
            pltpu.VMEM((EP if do_count else 8,), jnp.int32),
            pltpu.VMEM_SHARED((EP, D), _f32),
            pltpu.SemaphoreType.DMA,
        ],
        compiler_params=pltpu.CompilerParams(needs_layout_passes=False),
    )


def _mid_body(s_ref, c_ref, de_ref, o_ref):
    ssum = s_ref[0] + s_ref[1]
    cnt = jnp.maximum(
        jnp.sum(c_ref[...], axis=(0, 1)).astype(_f32), 1.0)[:, None]
    o_ref[...] = ssum / cnt * de_ref[...]


def _fin_body(s_ref, dv_ref, x0_ref, w_ref, ab_ref, o_ref):
    xv = (s_ref[0] + s_ref[1]) * dv_ref[...]
    nrm = jnp.sqrt(jnp.sum(xv * xv, axis=1, keepdims=True))
    xn = xv * jnp.where(nrm > 0.0, 1.0 / nrm, 0.0)
    a = ab_ref[0]
    b = ab_ref[1]
    xi = (1.0 - a) * xn + a * x0_ref[...]
    o_ref[...] = (1.0 - b) * xi + b * jnp.dot(
        xi, w_ref[...], preferred_element_type=jnp.float32)


_RB = 1024

_mid = pl.pallas_call(
    _mid_body,
    grid=(EP // _RB,),
    in_specs=[
        pl.BlockSpec((NC, _RB, D), lambda i: (0, i, 0)),
        pl.BlockSpec((NC, NS, _RB), lambda i: (0, 0, i)),
        pl.BlockSpec((_RB, 1), lambda i: (i, 0)),
    ],
    out_specs=pl.BlockSpec((_RB, D), lambda i: (i, 0)),
    out_shape=jax.ShapeDtypeStruct((EP, D), _f32),
)

_fin = pl.pallas_call(
    _fin_body,
    grid=(EP // _RB,),
    in_specs=[
        pl.BlockSpec((NC, _RB, D), lambda i: (0, i, 0)),
        pl.BlockSpec((_RB, 1), lambda i: (i, 0)),
        pl.BlockSpec((_RB, D), lambda i: (i, 0)),
        pl.BlockSpec((D, D), lambda i: (0, 0)),
        pl.BlockSpec(memory_space=pltpu.SMEM),
    ],
    out_specs=pl.BlockSpec((_RB, D), lambda i: (i, 0)),
    out_shape=jax.ShapeDtypeStruct((EP, D), _f32),
)


def kernel(X, vertex, edges, alpha, beta, X0, degE, degV, W):
    v32 = vertex.astype(jnp.int32)
    e32 = edges.astype(jnp.int32)
    pad = PAD_NNZ - NNZ
    sprd = jnp.arange(pad, dtype=jnp.int32)
    gpad = sprd % N
    spad = E + sprd % (EP - E)
    vg = jnp.concatenate([v32, gpad]).reshape(NCHUNKS, CH)
    es = jnp.concatenate([e32, spad]).reshape(NCHUNKS, CH)
    eg = jnp.concatenate([e32, gpad]).reshape(NCHUNKS, CH)
    vs = jnp.concatenate([v32, spad]).reshape(NCHUNKS, CH)
    zrow = jnp.zeros((CH, D), _f32)
    zhist = jnp.zeros((EP,), jnp.int32)
    degEp = jnp.concatenate([degE, jnp.ones((EP - E, 1), _f32)])
    degVp = jnp.concatenate([degV, jnp.ones((EP - N, 1), _f32)])
    X0p = jnp.concatenate([X0, jnp.zeros((EP - N, D), _f32)])
    ab = jnp.stack([jnp.asarray(alpha, _f32), jnp.asarray(beta, _f32)])

    s1, cnt1 = _build_hop(True)(X, vg, es, zrow, zhist)
    Xe = _mid(s1, cnt1, degEp)
    s2, _ = _build_hop(False)(Xe, eg, vs, zrow, zhist)
    out = _fin(s2, degVp, X0p, W, ab)
    return out[:N]
